# baseline (speedup 1.0000x reference)
import jax
import jax.numpy as jnp
from jax.experimental import pallas as pl
from jax.experimental.pallas import tpu as pltpu

_B, _S, _D = 4, 2048, 1024
_N = _B * _S
_C = 256
_NBUF = 8
_NCH = _N // _C
_PE_NCH = _S // _C


def _body(x_hbm, pe_hbm, o_hbm, xbuf, pebuf, obuf, insem, pesem, outsem):
    def xin(i):
        slot = i % _NBUF
        return pltpu.make_async_copy(
            x_hbm.at[pl.ds(i * _C, _C), :], xbuf.at[slot], insem.at[slot]
        )

    def pein(j):
        return pltpu.make_async_copy(
            pe_hbm.at[pl.ds(j * _C, _C), :], pebuf.at[pl.ds(j * _C, _C), :],
            pesem.at[j],
        )

    def oout(i):
        slot = i % _NBUF
        return pltpu.make_async_copy(
            obuf.at[slot], o_hbm.at[pl.ds(i * _C, _C), :], outsem.at[slot]
        )

    for j in range(_PE_NCH):
        pein(j).start()
    for i in range(_NBUF):
        xin(i).start()

    for i in range(_NCH):
        slot = i % _NBUF
        xin(i).wait()
        if i < _PE_NCH:
            pein(i).wait()
        if i >= _NBUF:
            oout(i - _NBUF).wait()
        poff = (i % _PE_NCH) * _C
        obuf[slot] = xbuf[slot] + pebuf[pl.ds(poff, _C), :]
        oout(i).start()
        if i + _NBUF < _NCH:
            xin(i + _NBUF).start()

    for i in range(_NCH - _NBUF, _NCH):
        oout(i).wait()


def kernel(x, pe_weight):
    out = pl.pallas_call(
        _body,
        in_specs=[
            pl.BlockSpec(memory_space=pltpu.MemorySpace.HBM),
            pl.BlockSpec(memory_space=pltpu.MemorySpace.HBM),
        ],
        out_specs=pl.BlockSpec(memory_space=pltpu.MemorySpace.HBM),
        out_shape=jax.ShapeDtypeStruct((_N, _D), x.dtype),
        scratch_shapes=[
            pltpu.VMEM((_NBUF, _C, _D), jnp.float32),
            pltpu.VMEM((_S, _D), jnp.float32),
            pltpu.VMEM((_NBUF, _C, _D), jnp.float32),
            pltpu.SemaphoreType.DMA((_NBUF,)),
            pltpu.SemaphoreType.DMA((_PE_NCH,)),
            pltpu.SemaphoreType.DMA((_NBUF,)),
        ],
    )(x.reshape(_N, _D), pe_weight)
    return out.reshape(x.shape)


# speedup vs baseline: 1.0133x; 1.0133x over previous
import jax
import jax.numpy as jnp
from jax.experimental import pallas as pl


_BS = 2048


def _add_body(x_ref, pe_ref, o_ref):
    o_ref[...] = x_ref[...] + pe_ref[...][None]


def kernel(x, pe_weight):
    B, S, D = x.shape
    grid = (S // _BS, B)
    return pl.pallas_call(
        _add_body,
        grid=grid,
        in_specs=[
            pl.BlockSpec((1, _BS, D), lambda s, b: (b, s, 0)),
            pl.BlockSpec((_BS, D), lambda s, b: (s, 0)),
        ],
        out_specs=pl.BlockSpec((1, _BS, D), lambda s, b: (b, s, 0)),
        out_shape=jax.ShapeDtypeStruct((B, S, D), x.dtype),
    )(x, pe_weight)
